# baseline (speedup 1.0000x reference)
import functools

import jax
import jax.numpy as jnp
from jax import lax
from jax.experimental import pallas as pl
from jax.experimental.pallas import tpu as pltpu
from jax.experimental.pallas import tpu_sc as plsc

NUM_EMB = 1000000
DIM = 64
WIDE = 128
BATCH = 16384
N_FIELDS = 26
TOTAL = BATCH * N_FIELDS
NUM_CORES = 2
NUM_SUBCORES = 16
NW = NUM_CORES * NUM_SUBCORES
PER_W = TOTAL // NW
CHUNK = 128
NCHUNK = PER_W // CHUNK
NBUF = 4
NGROUPS = NCHUNK // NBUF

assert TOTAL % NW == 0 and PER_W % CHUNK == 0 and NCHUNK % NBUF == 0

_mesh = plsc.VectorSubcoreMesh(core_axis_name="c", subcore_axis_name="s")


@functools.partial(
    pl.kernel,
    mesh=_mesh,
    out_type=jax.ShapeDtypeStruct((TOTAL, WIDE), jnp.float32),
    scratch_types=(
        [pltpu.VMEM((PER_W,), jnp.int32)]
        + [pltpu.VMEM((CHUNK, WIDE), jnp.float32) for _ in range(NBUF)]
        + [pltpu.SemaphoreType.DMA for _ in range(2 * NBUF)]
    ),
)
def _gather_kernel(weight_hbm, idx_hbm, out_hbm, idx_v, *bufs):
    rows = bufs[:NBUF]
    gsem = bufs[NBUF:2 * NBUF]
    ssem = bufs[2 * NBUF:]
    wid = lax.axis_index("s") * NUM_CORES + lax.axis_index("c")
    base = wid * PER_W
    pltpu.sync_copy(idx_hbm.at[pl.ds(base, PER_W)], idx_v)

    def gather_start(i, b):
        pltpu.async_copy(
            weight_hbm.at[idx_v.at[pl.ds(i * CHUNK, CHUNK)]], rows[b], gsem[b]
        )

    def gather_wait(b):
        pltpu.make_async_copy(
            weight_hbm.at[idx_v.at[pl.ds(0, CHUNK)]], rows[b], gsem[b]
        ).wait()

    def store_start(i, b):
        pltpu.async_copy(
            rows[b], out_hbm.at[pl.ds(base + i * CHUNK, CHUNK)], ssem[b]
        )

    def store_wait(b):
        pltpu.make_async_copy(
            rows[b], out_hbm.at[pl.ds(base, CHUNK)], ssem[b]
        ).wait()

    for b in range(NBUF):
        gather_start(b, b)

    def body(g, carry):
        for b in range(NBUF):
            gather_wait(b)
            store_start(g * NBUF + b, b)
        for b in range(NBUF):
            store_wait(b)
            gather_start((g + 1) * NBUF + b, b)
        return carry

    lax.fori_loop(0, NGROUPS - 1, body, 0)

    for b in range(NBUF):
        gather_wait(b)
        store_start((NGROUPS - 1) * NBUF + b, b)
    for b in range(NBUF):
        store_wait(b)


def kernel(idx, weight):
    flat_idx = idx.reshape(TOTAL).astype(jnp.int32)
    w_wide = jnp.tile(weight, (1, 2))
    out = _gather_kernel(w_wide, flat_idx)
    return out[:, :DIM].reshape(BATCH, N_FIELDS, DIM)


# speedup vs baseline: 1.4218x; 1.4218x over previous
import functools

import jax
import jax.numpy as jnp
from jax import lax
from jax.experimental import pallas as pl
from jax.experimental.pallas import tpu as pltpu
from jax.experimental.pallas import tpu_sc as plsc

NUM_EMB = 1000000
DIM = 64
BATCH = 16384
N_FIELDS = 26
TOTAL = BATCH * N_FIELDS
NUM_CORES = 2
NUM_SUBCORES = 16
NW = NUM_CORES * NUM_SUBCORES
PER_W = TOTAL // NW
CHUNK = 256
NCHUNK = PER_W // CHUNK
NBUF = 4
NGROUPS = NCHUNK // NBUF

BT = 1024
NBLK = -(-NUM_EMB // (2 * BT))
DENSE_ROWS = NBLK * BT
NUM_EMB_PAD = 2 * DENSE_ROWS
STD_LIMIT = (NBLK - 1) * 2 * BT
CLAMP_B = NUM_EMB - BT

assert TOTAL % NW == 0 and PER_W % CHUNK == 0 and NCHUNK % NBUF == 0

_mesh = plsc.VectorSubcoreMesh(core_axis_name="c", subcore_axis_name="s")


def _transpose_body(wt_ref, out_ref):
    x = wt_ref[...]
    out_ref[:, :DIM] = jnp.transpose(x[:, :BT], (1, 0))
    out_ref[:, DIM:] = jnp.transpose(x[:, BT:], (1, 0))


_transpose_kernel = pl.pallas_call(
    _transpose_body,
    grid=(NBLK,),
    in_specs=[pl.BlockSpec((DIM, 2 * BT), lambda k: (0, k))],
    out_specs=pl.BlockSpec((BT, 2 * DIM), lambda k: (k, 0)),
    out_shape=jax.ShapeDtypeStruct((DENSE_ROWS, 2 * DIM), jnp.float32),
)


@functools.partial(
    pl.kernel,
    mesh=_mesh,
    out_type=jax.ShapeDtypeStruct((TOTAL, DIM), jnp.float32),
    scratch_types=(
        [pltpu.VMEM((PER_W,), jnp.int32)]
        + [pltpu.VMEM((CHUNK, DIM), jnp.float32) for _ in range(NBUF)]
        + [pltpu.SemaphoreType.DMA for _ in range(2 * NBUF)]
    ),
    compiler_params=pltpu.CompilerParams(use_tc_tiling_on_sc=False),
)
def _gather_kernel(weight_hbm, idx_hbm, out_hbm, idx_v, *bufs):
    rows = bufs[:NBUF]
    gsem = bufs[NBUF:2 * NBUF]
    ssem = bufs[2 * NBUF:]
    wid = lax.axis_index("s") * NUM_CORES + lax.axis_index("c")
    base = wid * PER_W
    pltpu.sync_copy(idx_hbm.at[pl.ds(base, PER_W)], idx_v)

    def gather_start(i, b):
        pltpu.async_copy(
            weight_hbm.at[idx_v.at[pl.ds(i * CHUNK, CHUNK)]], rows[b], gsem[b]
        )

    def gather_wait(b):
        pltpu.make_async_copy(
            weight_hbm.at[idx_v.at[pl.ds(0, CHUNK)]], rows[b], gsem[b]
        ).wait()

    def store_start(i, b):
        pltpu.async_copy(
            rows[b], out_hbm.at[pl.ds(base + i * CHUNK, CHUNK)], ssem[b]
        )

    def store_wait(b):
        pltpu.make_async_copy(
            rows[b], out_hbm.at[pl.ds(base, CHUNK)], ssem[b]
        ).wait()

    for b in range(NBUF):
        gather_start(b, b)

    def body(g, carry):
        for b in range(NBUF):
            gather_wait(b)
            store_start(g * NBUF + b, b)
        for b in range(NBUF):
            store_wait(b)
            gather_start((g + 1) * NBUF + b, b)
        return carry

    lax.fori_loop(0, NGROUPS - 1, body, 0)

    for b in range(NBUF):
        gather_wait(b)
        store_start((NGROUPS - 1) * NBUF + b, b)
    for b in range(NBUF):
        store_wait(b)


def kernel(idx, weight):
    flat = idx.reshape(TOTAL).astype(jnp.int32)
    std = (
        (flat // (2 * BT)) * (2 * BT)
        + (flat % BT) * 2
        + (flat // BT) % 2
    )
    flat_idx = std
    w_dense = _transpose_kernel(weight.T)
    w3 = w_dense.reshape(NUM_EMB_PAD, DIM)
    out = _gather_kernel(w3, flat_idx)
    return out.reshape(BATCH, N_FIELDS, DIM)


# speedup vs baseline: 1.4252x; 1.0024x over previous
import functools

import jax
import jax.numpy as jnp
from jax import lax
from jax.experimental import pallas as pl
from jax.experimental.pallas import tpu as pltpu
from jax.experimental.pallas import tpu_sc as plsc

NUM_EMB = 1000000
DIM = 64
BATCH = 16384
N_FIELDS = 26
TOTAL = BATCH * N_FIELDS
NUM_CORES = 2
NUM_SUBCORES = 16
NW = NUM_CORES * NUM_SUBCORES
PER_W = TOTAL // NW
CHUNK = 256
NCHUNK = PER_W // CHUNK
NBUF = 4
NGROUPS = NCHUNK // NBUF

BT = 1024
NBLK = -(-NUM_EMB // (2 * BT))
DENSE_ROWS = NBLK * BT
NUM_EMB_PAD = 2 * DENSE_ROWS

assert TOTAL % NW == 0 and PER_W % CHUNK == 0 and NCHUNK % NBUF == 0

_mesh = plsc.VectorSubcoreMesh(core_axis_name="c", subcore_axis_name="s")


def _transpose_body(wt_ref, out_ref):
    x = wt_ref[...]
    out_ref[:, :DIM] = jnp.transpose(x[:, :BT], (1, 0))
    out_ref[:, DIM:] = jnp.transpose(x[:, BT:], (1, 0))


_transpose_kernel = pl.pallas_call(
    _transpose_body,
    grid=(NBLK,),
    in_specs=[pl.BlockSpec((DIM, 2 * BT), lambda k: (0, k))],
    out_specs=pl.BlockSpec((BT, 2 * DIM), lambda k: (k, 0)),
    out_shape=jax.ShapeDtypeStruct((DENSE_ROWS, 2 * DIM), jnp.float32),
)


@functools.partial(
    pl.kernel,
    mesh=_mesh,
    out_type=jax.ShapeDtypeStruct((TOTAL, DIM), jnp.float32),
    scratch_types=(
        [pltpu.VMEM((PER_W,), jnp.int32)]
        + [pltpu.VMEM((CHUNK, DIM), jnp.float32) for _ in range(NBUF)]
        + [pltpu.SemaphoreType.DMA for _ in range(2 * NBUF)]
    ),
    compiler_params=pltpu.CompilerParams(use_tc_tiling_on_sc=False),
)
def _gather_kernel(weight_hbm, idx_hbm, out_hbm, idx_v, *bufs):
    rows = bufs[:NBUF]
    gsem = bufs[NBUF:2 * NBUF]
    ssem = bufs[2 * NBUF:]
    wid = lax.axis_index("s") * NUM_CORES + lax.axis_index("c")
    base = wid * PER_W
    pltpu.sync_copy(idx_hbm.at[pl.ds(base, PER_W)], idx_v)

    def gather_start(i, b):
        pltpu.async_copy(
            weight_hbm.at[idx_v.at[pl.ds(i * CHUNK, CHUNK)]], rows[b], gsem[b]
        )

    def gather_wait(b):
        pltpu.make_async_copy(
            weight_hbm.at[idx_v.at[pl.ds(0, CHUNK)]], rows[b], gsem[b]
        ).wait()

    def store_start(i, b):
        pltpu.async_copy(
            rows[b], out_hbm.at[pl.ds(base + i * CHUNK, CHUNK)], ssem[b]
        )

    def store_wait(b):
        pltpu.make_async_copy(
            rows[b], out_hbm.at[pl.ds(base, CHUNK)], ssem[b]
        ).wait()

    for b in range(NBUF):
        gather_start(b, b)

    def body(g, carry):
        for b in range(NBUF):
            gather_wait(b)
            store_start(g * NBUF + b, b)
        for b in range(NBUF):
            store_wait(b)
            gather_start((g + 1) * NBUF + b, b)
        return carry

    lax.fori_loop(0, NGROUPS - 1, body, 0)

    for b in range(NBUF):
        gather_wait(b)
        store_start((NGROUPS - 1) * NBUF + b, b)
    for b in range(NBUF):
        store_wait(b)


def kernel(idx, weight):
    flat = idx.reshape(TOTAL).astype(jnp.int32)
    std = (
        (flat // (2 * BT)) * (2 * BT)
        + (flat % BT) * 2
        + (flat // BT) % 2
    )
    flat_idx = std
    w_dense = _transpose_kernel(weight.T)
    w3 = w_dense.reshape(NUM_EMB_PAD, DIM)
    out = _gather_kernel(w3, flat_idx)
    return out.reshape(BATCH, N_FIELDS, DIM)
